# baseline (device time: 48091 ns/iter reference)
import jax
import jax.numpy as jnp
from jax import lax
from jax.experimental import pallas as pl
from jax.experimental.pallas import tpu as pltpu

N_DEV = 4
E_PER = 2


def kernel(x, assign, W1, W2):
    t, d = x.shape
    e_per, _, f = W1.shape
    assert e_per == E_PER
    a2 = assign.reshape(t, 1)

    def body(x_ref, a_ref, w1_ref, w2_ref, out_ref,
             xsend, xbuf, abuf, psend, rbuf,
             xs_sems, as_sems, rs_sems, xr_sems, ar_sems, rr_sems):
        me = lax.axis_index("i")

        barrier = pltpu.get_barrier_semaphore()
        for j in range(1, N_DEV):
            p = lax.rem(me + j, N_DEV)
            pl.semaphore_signal(barrier, inc=1, device_id=(p,),
                                device_id_type=pl.DeviceIdType.MESH)
        pl.semaphore_wait(barrier, N_DEV - 1)

        xsend[...] = x_ref[...].astype(jnp.bfloat16)

        sends = []
        for j in range(1, N_DEV):
            p = lax.rem(me + j, N_DEV)
            slot = N_DEV - j - 1
            rx = pltpu.make_async_remote_copy(
                src_ref=xsend, dst_ref=xbuf.at[slot],
                send_sem=xs_sems.at[j - 1], recv_sem=xr_sems.at[slot],
                device_id=(p,), device_id_type=pl.DeviceIdType.MESH)
            ra = pltpu.make_async_remote_copy(
                src_ref=a_ref, dst_ref=abuf.at[slot],
                send_sem=as_sems.at[j - 1], recv_sem=ar_sems.at[slot],
                device_id=(p,), device_id_type=pl.DeviceIdType.MESH)
            rx.start()
            ra.start()
            sends += [rx, ra]

        w1 = [w1_ref[k].astype(jnp.bfloat16) for k in range(E_PER)]
        w2 = [w2_ref[k].astype(jnp.bfloat16) for k in range(E_PER)]

        def apply_experts(xc, ac):
            acc = None
            for k in range(E_PER):
                e = E_PER * me + k
                xm = jnp.where(ac == e, xc, jnp.zeros_like(xc))
                h = jnp.dot(xm, w1[k], preferred_element_type=jnp.float32)
                h = jnp.maximum(h, 0.0).astype(jnp.bfloat16)
                y = jnp.dot(h, w2[k], preferred_element_type=jnp.float32)
                acc = y if acc is None else acc + y
            return acc

        out_ref[...] = apply_experts(xsend[...], a_ref[...])

        for q in range(1, N_DEV):
            slot = q - 1
            pltpu.make_async_remote_copy(
                src_ref=xsend, dst_ref=xbuf.at[slot],
                send_sem=xs_sems.at[0], recv_sem=xr_sems.at[slot],
                device_id=(me,), device_id_type=pl.DeviceIdType.MESH,
            ).wait_recv()
            pltpu.make_async_remote_copy(
                src_ref=a_ref, dst_ref=abuf.at[slot],
                send_sem=as_sems.at[0], recv_sem=ar_sems.at[slot],
                device_id=(me,), device_id_type=pl.DeviceIdType.MESH,
            ).wait_recv()
            psend[slot] = apply_experts(xbuf[slot], abuf[slot]).astype(
                jnp.bfloat16)
            owner = lax.rem(me + q, N_DEV)
            rslot = N_DEV - q - 1
            rr = pltpu.make_async_remote_copy(
                src_ref=psend.at[slot], dst_ref=rbuf.at[rslot],
                send_sem=rs_sems.at[slot], recv_sem=rr_sems.at[rslot],
                device_id=(owner,), device_id_type=pl.DeviceIdType.MESH)
            rr.start()
            sends.append(rr)

        for i in range(N_DEV - 1):
            pltpu.make_async_remote_copy(
                src_ref=psend.at[0], dst_ref=rbuf.at[i],
                send_sem=rs_sems.at[0], recv_sem=rr_sems.at[i],
                device_id=(me,), device_id_type=pl.DeviceIdType.MESH,
            ).wait_recv()
        out_ref[...] = (out_ref[...]
                        + rbuf[0].astype(jnp.float32)
                        + rbuf[1].astype(jnp.float32)
                        + rbuf[2].astype(jnp.float32))

        for dsc in sends:
            dsc.wait_send()

    return pl.pallas_call(
        body,
        out_shape=jax.ShapeDtypeStruct((t, d), jnp.float32),
        in_specs=[pl.BlockSpec(memory_space=pltpu.VMEM)] * 4,
        out_specs=pl.BlockSpec(memory_space=pltpu.VMEM),
        scratch_shapes=[
            pltpu.VMEM((t, d), jnp.bfloat16),
            pltpu.VMEM((N_DEV - 1, t, d), jnp.bfloat16),
            pltpu.VMEM((N_DEV - 1, t, 1), jnp.int32),
            pltpu.VMEM((N_DEV - 1, t, d), jnp.bfloat16),
            pltpu.VMEM((N_DEV - 1, t, d), jnp.bfloat16),
            pltpu.SemaphoreType.DMA((N_DEV - 1,)),
            pltpu.SemaphoreType.DMA((N_DEV - 1,)),
            pltpu.SemaphoreType.DMA((N_DEV - 1,)),
            pltpu.SemaphoreType.DMA((N_DEV - 1,)),
            pltpu.SemaphoreType.DMA((N_DEV - 1,)),
            pltpu.SemaphoreType.DMA((N_DEV - 1,)),
        ],
        compiler_params=pltpu.CompilerParams(collective_id=0),
    )(x, a2, W1, W2)


# device time: 30561 ns/iter; 1.5736x vs baseline; 1.5736x over previous
import jax
import jax.numpy as jnp
from jax import lax
from jax.experimental import pallas as pl
from jax.experimental.pallas import tpu as pltpu

N_DEV = 4
E_PER = 2
CAP = 192


def kernel(x, assign, W1, W2):
    t, d = x.shape
    e_per, _, f = W1.shape
    assert e_per == E_PER
    a_row = assign.reshape(1, t)
    a_col = assign.reshape(t, 1)

    def body(x_ref, a_ref, ac_ref, w1_ref, w2_ref, out_ref,
             xsend, asend, psend, xbuf, abuf, rbuf,
             xs_sems, as_sems, rs_sems, xr_sems, ar_sems, rr_sems):
        me = lax.axis_index("i")

        barrier = pltpu.get_barrier_semaphore()
        for j in range(1, N_DEV):
            p = lax.rem(me + j, N_DEV)
            pl.semaphore_signal(barrier, inc=1, device_id=(p,),
                                device_id_type=pl.DeviceIdType.MESH)
        pl.semaphore_wait(barrier, N_DEV - 1)

        xb = x_ref[...].astype(jnp.bfloat16)
        pair_row = a_ref[...] // E_PER
        a1_col = (ac_ref[...] + 1).astype(jnp.bfloat16)

        ii = lax.broadcasted_iota(jnp.int32, (t, t), 0)
        jj = lax.broadcasted_iota(jnp.int32, (t, t), 1)
        U = (ii < jj).astype(jnp.bfloat16)
        iota_c = lax.broadcasted_iota(jnp.int32, (CAP, 1), 0)

        P = []
        xc = []
        ac = []
        for j in range(N_DEV):
            dest = lax.rem(me + j, N_DEV)
            m_row = pair_row == dest
            cs = jnp.dot(m_row.astype(jnp.bfloat16), U,
                         preferred_element_type=jnp.float32
                         ).astype(jnp.int32)
            Pj = ((cs == iota_c) & m_row).astype(jnp.bfloat16)
            P.append(Pj)
            xc.append(jnp.dot(Pj, xb,
                              preferred_element_type=jnp.float32
                              ).astype(jnp.bfloat16))
            ac.append(jnp.dot(Pj, a1_col,
                              preferred_element_type=jnp.float32))

        sends = []
        for j in range(1, N_DEV):
            p = lax.rem(me + j, N_DEV)
            slot = N_DEV - j - 1
            xsend[j - 1] = xc[j]
            asend[j - 1] = ac[j]
            rx = pltpu.make_async_remote_copy(
                src_ref=xsend.at[j - 1], dst_ref=xbuf.at[slot],
                send_sem=xs_sems.at[j - 1], recv_sem=xr_sems.at[slot],
                device_id=(p,), device_id_type=pl.DeviceIdType.MESH)
            ra = pltpu.make_async_remote_copy(
                src_ref=asend.at[j - 1], dst_ref=abuf.at[slot],
                send_sem=as_sems.at[j - 1], recv_sem=ar_sems.at[slot],
                device_id=(p,), device_id_type=pl.DeviceIdType.MESH)
            rx.start()
            ra.start()
            sends += [rx, ra]

        w1 = [w1_ref[k].astype(jnp.bfloat16) for k in range(E_PER)]
        w2 = [w2_ref[k].astype(jnp.bfloat16) for k in range(E_PER)]

        def apply_experts(xg, ag):
            acc = None
            for k in range(E_PER):
                e1 = (E_PER * me + k + 1).astype(jnp.float32)
                xm = jnp.where(ag == e1, xg, jnp.zeros_like(xg))
                h = jnp.dot(xm, w1[k], preferred_element_type=jnp.float32)
                h = jnp.maximum(h, 0.0).astype(jnp.bfloat16)
                y = jnp.dot(h, w2[k], preferred_element_type=jnp.float32)
                acc = y if acc is None else acc + y
            return acc

        y_own = apply_experts(xc[0], ac[0])

        for i in range(N_DEV - 1):
            pltpu.make_async_remote_copy(
                src_ref=xsend.at[0], dst_ref=xbuf.at[i],
                send_sem=xs_sems.at[0], recv_sem=xr_sems.at[i],
                device_id=(me,), device_id_type=pl.DeviceIdType.MESH,
            ).wait_recv()
            pltpu.make_async_remote_copy(
                src_ref=asend.at[0], dst_ref=abuf.at[i],
                send_sem=as_sems.at[0], recv_sem=ar_sems.at[i],
                device_id=(me,), device_id_type=pl.DeviceIdType.MESH,
            ).wait_recv()
            psend[i] = apply_experts(xbuf[i], abuf[i]).astype(jnp.bfloat16)
            owner = lax.rem(me + i + 1, N_DEV)
            rr = pltpu.make_async_remote_copy(
                src_ref=psend.at[i], dst_ref=rbuf.at[2 - i],
                send_sem=rs_sems.at[i], recv_sem=rr_sems.at[2 - i],
                device_id=(owner,), device_id_type=pl.DeviceIdType.MESH)
            rr.start()
            sends.append(rr)

        out = lax.dot_general(P[0], y_own.astype(jnp.bfloat16),
                              (((0,), (0,)), ((), ())),
                              preferred_element_type=jnp.float32)
        for k in range(N_DEV - 1):
            pltpu.make_async_remote_copy(
                src_ref=psend.at[0], dst_ref=rbuf.at[k],
                send_sem=rs_sems.at[0], recv_sem=rr_sems.at[k],
                device_id=(me,), device_id_type=pl.DeviceIdType.MESH,
            ).wait_recv()
            out = out + lax.dot_general(
                P[k + 1], rbuf[k], (((0,), (0,)), ((), ())),
                preferred_element_type=jnp.float32)
        out_ref[...] = out

        for dsc in sends:
            dsc.wait_send()

    return pl.pallas_call(
        body,
        out_shape=jax.ShapeDtypeStruct((t, d), jnp.float32),
        in_specs=[pl.BlockSpec(memory_space=pltpu.VMEM)] * 5,
        out_specs=pl.BlockSpec(memory_space=pltpu.VMEM),
        scratch_shapes=[
            pltpu.VMEM((N_DEV - 1, CAP, d), jnp.bfloat16),
            pltpu.VMEM((N_DEV - 1, CAP, 1), jnp.float32),
            pltpu.VMEM((N_DEV - 1, CAP, d), jnp.bfloat16),
            pltpu.VMEM((N_DEV - 1, CAP, d), jnp.bfloat16),
            pltpu.VMEM((N_DEV - 1, CAP, 1), jnp.float32),
            pltpu.VMEM((N_DEV - 1, CAP, d), jnp.bfloat16),
            pltpu.SemaphoreType.DMA((N_DEV - 1,)),
            pltpu.SemaphoreType.DMA((N_DEV - 1,)),
            pltpu.SemaphoreType.DMA((N_DEV - 1,)),
            pltpu.SemaphoreType.DMA((N_DEV - 1,)),
            pltpu.SemaphoreType.DMA((N_DEV - 1,)),
            pltpu.SemaphoreType.DMA((N_DEV - 1,)),
        ],
        compiler_params=pltpu.CompilerParams(collective_id=0),
    )(x, a_row, a_col, W1, W2)


# device time: 25949 ns/iter; 1.8533x vs baseline; 1.1777x over previous
import jax
import jax.numpy as jnp
from jax import lax
from jax.experimental import pallas as pl
from jax.experimental.pallas import tpu as pltpu

N_DEV = 4
E_PER = 2
CAP_E = 96
CAP = E_PER * CAP_E


def kernel(x, assign, W1, W2):
    t, d = x.shape
    e_per, _, f = W1.shape
    assert e_per == E_PER
    a_row = assign.reshape(1, t)
    a_col = assign.reshape(t, 1)

    def body(x_ref, a_ref, ac_ref, w1_ref, w2_ref, out_ref,
             xsend, psend, xbuf, rbuf,
             xs_sems, rs_sems, xr_sems, rr_sems):
        me = lax.axis_index("i")

        barrier = pltpu.get_barrier_semaphore()
        for j in range(1, N_DEV):
            p = lax.rem(me + j, N_DEV)
            pl.semaphore_signal(barrier, inc=1, device_id=(p,),
                                device_id_type=pl.DeviceIdType.MESH)
        pl.semaphore_wait(barrier, N_DEV - 1)

        xb = x_ref[...].astype(jnp.bfloat16)
        arow = a_ref[...]
        acol = ac_ref[...]

        ii = lax.broadcasted_iota(jnp.int32, (t, t), 0)
        jj = lax.broadcasted_iota(jnp.int32, (t, t), 1)
        U = (ii < jj).astype(jnp.bfloat16)
        L = (ii > jj).astype(jnp.bfloat16)
        iota_c = lax.broadcasted_iota(jnp.int32, (CAP, 1), 0)
        iota_r = lax.broadcasted_iota(jnp.int32, (1, CAP), 1)

        def perm_pair(dest):
            e0 = E_PER * dest
            m0r, m1r = arow == e0, arow == e0 + 1
            m0c, m1c = acol == e0, acol == e0 + 1
            m2 = jnp.concatenate(
                [m0r.astype(jnp.bfloat16), m1r.astype(jnp.bfloat16)], axis=0)
            cs2 = jnp.dot(m2, U, preferred_element_type=jnp.float32
                          ).astype(jnp.int32)
            m2c = jnp.concatenate(
                [m0c.astype(jnp.bfloat16), m1c.astype(jnp.bfloat16)], axis=1)
            cs2c = jnp.dot(L, m2c, preferred_element_type=jnp.float32
                           ).astype(jnp.int32)
            P = (((cs2[0:1, :] == iota_c) & m0r)
                 | ((cs2[1:2, :] == iota_c - CAP_E) & m1r)
                 ).astype(jnp.bfloat16)
            Pt = (((cs2c[:, 0:1] == iota_r) & m0c)
                  | ((cs2c[:, 1:2] == iota_r - CAP_E) & m1c)
                  ).astype(jnp.bfloat16)
            return P, Pt

        Pt_all = [None] * N_DEV
        xc_own = None
        sends = []
        for j in (1, 2, 3, 0):
            dest = lax.rem(me + j, N_DEV)
            P, Pt = perm_pair(dest)
            Pt_all[j] = Pt
            xc = jnp.dot(P, xb, preferred_element_type=jnp.float32
                         ).astype(jnp.bfloat16)
            if j == 0:
                xc_own = xc
                continue
            slot = N_DEV - j - 1
            xsend[j - 1] = xc
            rx = pltpu.make_async_remote_copy(
                src_ref=xsend.at[j - 1], dst_ref=xbuf.at[slot],
                send_sem=xs_sems.at[j - 1], recv_sem=xr_sems.at[slot],
                device_id=(lax.rem(me + j, N_DEV),),
                device_id_type=pl.DeviceIdType.MESH)
            rx.start()
            sends.append(rx)

        W1cat = jnp.concatenate(
            [w1_ref[0].astype(jnp.bfloat16), w1_ref[1].astype(jnp.bfloat16)],
            axis=1)
        W2cat = w2_ref[...].astype(jnp.bfloat16).reshape(E_PER * f, d)
        hr = lax.broadcasted_iota(jnp.int32, (CAP, 1), 0) // CAP_E
        hc = lax.broadcasted_iota(jnp.int32, (1, E_PER * f), 1) // f
        HM = hr == hc

        def apply_group(xg):
            h = jnp.dot(xg, W1cat, preferred_element_type=jnp.float32)
            hb = jnp.maximum(h, 0.0).astype(jnp.bfloat16)
            hm = jnp.where(HM, hb, jnp.zeros_like(hb))
            return jnp.dot(hm, W2cat, preferred_element_type=jnp.float32)

        y_own = apply_group(xc_own).astype(jnp.bfloat16)

        for i in range(N_DEV - 1):
            pltpu.make_async_remote_copy(
                src_ref=xsend.at[0], dst_ref=xbuf.at[i],
                send_sem=xs_sems.at[0], recv_sem=xr_sems.at[i],
                device_id=(me,), device_id_type=pl.DeviceIdType.MESH,
            ).wait_recv()
            psend[i] = apply_group(xbuf[i]).astype(jnp.bfloat16)
            owner = lax.rem(me + i + 1, N_DEV)
            rr = pltpu.make_async_remote_copy(
                src_ref=psend.at[i], dst_ref=rbuf.at[2 - i],
                send_sem=rs_sems.at[i], recv_sem=rr_sems.at[2 - i],
                device_id=(owner,), device_id_type=pl.DeviceIdType.MESH)
            rr.start()
            sends.append(rr)

        out = jnp.dot(Pt_all[0], y_own, preferred_element_type=jnp.float32)
        for k in range(N_DEV - 1):
            pltpu.make_async_remote_copy(
                src_ref=psend.at[0], dst_ref=rbuf.at[k],
                send_sem=rs_sems.at[0], recv_sem=rr_sems.at[k],
                device_id=(me,), device_id_type=pl.DeviceIdType.MESH,
            ).wait_recv()
            out = out + jnp.dot(Pt_all[k + 1], rbuf[k],
                                preferred_element_type=jnp.float32)
        out_ref[...] = out

        for dsc in sends:
            dsc.wait_send()

    return pl.pallas_call(
        body,
        out_shape=jax.ShapeDtypeStruct((t, d), jnp.float32),
        in_specs=[pl.BlockSpec(memory_space=pltpu.VMEM)] * 5,
        out_specs=pl.BlockSpec(memory_space=pltpu.VMEM),
        scratch_shapes=[
            pltpu.VMEM((N_DEV - 1, CAP, d), jnp.bfloat16),
            pltpu.VMEM((N_DEV - 1, CAP, d), jnp.bfloat16),
            pltpu.VMEM((N_DEV - 1, CAP, d), jnp.bfloat16),
            pltpu.VMEM((N_DEV - 1, CAP, d), jnp.bfloat16),
            pltpu.SemaphoreType.DMA((N_DEV - 1,)),
            pltpu.SemaphoreType.DMA((N_DEV - 1,)),
            pltpu.SemaphoreType.DMA((N_DEV - 1,)),
            pltpu.SemaphoreType.DMA((N_DEV - 1,)),
        ],
        compiler_params=pltpu.CompilerParams(collective_id=0),
    )(x, a_row, a_col, W1, W2)
